# baseline (speedup 1.0000x reference)
import functools

import jax
import jax.numpy as jnp
from jax import lax
from jax.experimental import pallas as pl
from jax.experimental.pallas import tpu as pltpu
from jax.experimental.pallas import tpu_sc as plsc

B = 16384
D = 32
IDX_CHUNK = 128
PART_OFF = (0, 249856, 499712, 749568)
PART_SZ = (249856, 249856, 249856, 250432)
P = len(PART_OFF)


@functools.lru_cache(maxsize=None)
def _make_gather(num_cores, num_subcores):
    NC, NS = num_cores, num_subcores
    NW = NC * NS
    b_per_w = B // NW
    n_chunks = b_per_w // IDX_CHUNK
    mesh = plsc.VectorSubcoreMesh(core_axis_name="c", subcore_axis_name="s")

    @functools.partial(
        pl.kernel,
        mesh=mesh,
        compiler_params=pltpu.CompilerParams(use_tc_tiling_on_sc=False),
        out_type=[
            jax.ShapeDtypeStruct((B, D), jnp.float32),
            jax.ShapeDtypeStruct((B, D), jnp.float32),
        ],
        scratch_types=[
            pltpu.VMEM((b_per_w,), jnp.int32),
            pltpu.VMEM((b_per_w,), jnp.int32),
            pltpu.VMEM((P, IDX_CHUNK), jnp.int32),
            pltpu.VMEM((P, IDX_CHUNK, D), jnp.float32),
            pltpu.VMEM((b_per_w, D), jnp.float32),
            pltpu.VMEM((b_per_w, D), jnp.float32),
            pltpu.SemaphoreType.DMA,
            pltpu.SemaphoreType.DMA,
        ],
    )
    def gather_k(up0, up1, up2, up3, movie_t, uid, mid, out_u, out_m,
                 uidx, midx, widx, prow, urows, mrows, usem, msem):
        uparts = (up0, up1, up2, up3)
        wid = lax.axis_index("s") * NC + lax.axis_index("c")
        base = wid * b_per_w
        pltpu.sync_copy(uid.at[pl.ds(base, b_per_w)], uidx)
        pltpu.sync_copy(mid.at[pl.ds(base, b_per_w)], midx)

        def chunk_body(k, _):
            o = k * IDX_CHUNK
            for p in range(P):
                for v in range(IDX_CHUNK // 16):
                    sl = pl.ds(o + v * 16, 16)
                    ids = uidx[sl]
                    inr = (ids >= PART_OFF[p]) & (ids < PART_OFF[p] + PART_SZ[p])
                    spread = (lax.broadcasted_iota(jnp.int32, (16,), 0)
                              + v * 16)
                    widx[p, pl.ds(v * 16, 16)] = jnp.where(
                        inr, ids - PART_OFF[p], spread)
            copies = [pltpu.make_async_copy(
                uparts[p].at[widx.at[p]], prow.at[p], usem)
                for p in range(P)]
            cm = pltpu.make_async_copy(
                movie_t.at[midx.at[pl.ds(o, IDX_CHUNK)]],
                mrows.at[pl.ds(o, IDX_CHUNK)], msem)
            for c in copies:
                c.start()
            cm.start()
            for c in copies:
                c.wait()
            for v in range(IDX_CHUNK // 16):
                ids = uidx[pl.ds(o + v * 16, 16)]
                for j in range(16):
                    r = ids[j]
                    i = v * 16 + j
                    for p in range(P):
                        cond = ((r >= PART_OFF[p])
                                & (r < PART_OFF[p] + PART_SZ[p]))

                        @pl.when(cond)
                        def _(p=p, i=i):
                            urows[o + i, pl.ds(0, 16)] = prow[p, i,
                                                              pl.ds(0, 16)]
                            urows[o + i, pl.ds(16, 16)] = prow[p, i,
                                                               pl.ds(16, 16)]
            cm.wait()
            return 0

        lax.fori_loop(0, n_chunks, chunk_body, 0)
        pltpu.sync_copy(urows, out_u.at[pl.ds(base, b_per_w)])
        pltpu.sync_copy(mrows, out_m.at[pl.ds(base, b_per_w)])

    return gather_k


BLK = 2048


def _mlp_body(u_ref, m_ref, c_ref, g_ref, a_ref, o_ref,
              gt_ref, at_ref, ot_ref,
              w0_ref, b0_ref, w1_ref, b1_ref, w2_ref, b2_ref,
              w3_ref, b3_ref, out_ref):
    f32 = jnp.float32
    acc = jnp.dot(u_ref[...], w0_ref[0:32, :], preferred_element_type=f32)
    acc += jnp.dot(m_ref[...], w0_ref[32:64, :], preferred_element_type=f32)
    acc += jnp.dot(c_ref[...], w0_ref[88:104, :], preferred_element_type=f32)

    def small(idx_ref, tab_ref, lo, hi, T):
        oh = (idx_ref[...] ==
              lax.broadcasted_iota(jnp.int32, (BLK, T), 1)).astype(f32)
        e = jnp.dot(oh, tab_ref[...], preferred_element_type=f32)
        return jnp.dot(e, w0_ref[lo:hi, :], preferred_element_type=f32)

    acc += small(g_ref, gt_ref, 64, 72, 2)
    acc += small(a_ref, at_ref, 72, 80, 7)
    acc += small(o_ref, ot_ref, 80, 88, 21)
    h = jnp.maximum(acc + b0_ref[...], 0.0)
    h = jnp.maximum(jnp.dot(h, w1_ref[...], preferred_element_type=f32)
                    + b1_ref[...], 0.0)
    h = jnp.maximum(jnp.dot(h, w2_ref[...], preferred_element_type=f32)
                    + b2_ref[...], 0.0)
    out_ref[...] = (jnp.dot(h, w3_ref[...], preferred_element_type=f32)
                    + b3_ref[...])


def _full(shape):
    return pl.BlockSpec(shape, lambda i: (0, 0))


_mlp_call = pl.pallas_call(
    _mlp_body,
    grid=(B // BLK,),
    in_specs=[
        pl.BlockSpec((BLK, D), lambda i: (i, 0)),
        pl.BlockSpec((BLK, D), lambda i: (i, 0)),
        pl.BlockSpec((BLK, 16), lambda i: (i, 0)),
        pl.BlockSpec((BLK, 1), lambda i: (i, 0)),
        pl.BlockSpec((BLK, 1), lambda i: (i, 0)),
        pl.BlockSpec((BLK, 1), lambda i: (i, 0)),
        _full((2, 8)), _full((7, 8)), _full((21, 8)),
        _full((104, 128)), _full((1, 128)),
        _full((128, 64)), _full((1, 64)),
        _full((64, 32)), _full((1, 32)),
        _full((32, 1)), _full((1, 1)),
    ],
    out_specs=pl.BlockSpec((BLK, 1), lambda i: (i, 0)),
    out_shape=jax.ShapeDtypeStruct((B, 1), jnp.float32),
)


def kernel(user_id, movie_id, gender, age, occupation, continuous_features,
           user_table, movie_table, gender_table, age_table, occupation_table,
           W0, b0, W1, b1, W2, b2, W3, b3):
    info = plsc.get_sparse_core_info()
    U, M = _make_gather(info.num_cores, info.num_subcores)(
        user_table[PART_OFF[0]:PART_OFF[0] + PART_SZ[0]],
        user_table[PART_OFF[1]:PART_OFF[1] + PART_SZ[1]],
        user_table[PART_OFF[2]:PART_OFF[2] + PART_SZ[2]],
        user_table[PART_OFF[3]:PART_OFF[3] + PART_SZ[3]],
        movie_table,
        user_id.astype(jnp.int32), movie_id.astype(jnp.int32))
    return _mlp_call(
        U, M, continuous_features,
        gender.astype(jnp.int32).reshape(B, 1),
        age.astype(jnp.int32).reshape(B, 1),
        occupation.astype(jnp.int32).reshape(B, 1),
        gender_table, age_table, occupation_table,
        W0, b0.reshape(1, 128), W1, b1.reshape(1, 64),
        W2, b2.reshape(1, 32), W3, b3.reshape(1, 1))


# speedup vs baseline: 1.3503x; 1.3503x over previous
import functools

import jax
import jax.numpy as jnp
from jax import lax
from jax.experimental import pallas as pl
from jax.experimental.pallas import tpu as pltpu
from jax.experimental.pallas import tpu_sc as plsc

B = 16384
D = 32
IDX_CHUNK = 128


@functools.lru_cache(maxsize=None)
def _make_gather(num_cores, num_subcores):
    NC, NS = num_cores, num_subcores
    NW = NC * NS
    b_per_w = B // NW
    n_chunks = b_per_w // IDX_CHUNK
    mesh = plsc.VectorSubcoreMesh(core_axis_name="c", subcore_axis_name="s")

    @functools.partial(
        pl.kernel,
        mesh=mesh,
        compiler_params=pltpu.CompilerParams(use_tc_tiling_on_sc=False),
        out_type=[
            jax.ShapeDtypeStruct((B, D), jnp.float32),
            jax.ShapeDtypeStruct((B, D), jnp.float32),
        ],
        scratch_types=[
            pltpu.VMEM((b_per_w,), jnp.int32),
            pltpu.VMEM((b_per_w,), jnp.int32),
            pltpu.VMEM((b_per_w, D), jnp.float32),
            pltpu.VMEM((b_per_w, D), jnp.float32),
            pltpu.SemaphoreType.DMA,
            pltpu.SemaphoreType.DMA,
        ],
    )
    def gather_k(user_t, movie_t, uid, mid, out_u, out_m,
                 uidx, midx, urows, mrows, usem, msem):
        wid = lax.axis_index("s") * NC + lax.axis_index("c")
        base = wid * b_per_w
        pltpu.sync_copy(uid.at[pl.ds(base, b_per_w)], uidx)
        pltpu.sync_copy(mid.at[pl.ds(base, b_per_w)], midx)
        copies = []
        for j in range(n_chunks):
            sl = pl.ds(j * IDX_CHUNK, IDX_CHUNK)
            copies.append(pltpu.make_async_copy(
                user_t.at[uidx.at[sl]], urows.at[sl], usem))
            copies.append(pltpu.make_async_copy(
                movie_t.at[midx.at[sl]], mrows.at[sl], msem))
        for c in copies:
            c.start()
        for c in copies:
            c.wait()
        pltpu.sync_copy(urows, out_u.at[pl.ds(base, b_per_w)])
        pltpu.sync_copy(mrows, out_m.at[pl.ds(base, b_per_w)])

    return gather_k


BLK = 2048


def _mlp_body(u_ref, m_ref, c_ref, g_ref, a_ref, o_ref,
              gt_ref, at_ref, ot_ref,
              w0_ref, b0_ref, w1_ref, b1_ref, w2_ref, b2_ref,
              w3_ref, b3_ref, out_ref):
    f32 = jnp.float32
    acc = jnp.dot(u_ref[...], w0_ref[0:32, :], preferred_element_type=f32)
    acc += jnp.dot(m_ref[...], w0_ref[32:64, :], preferred_element_type=f32)
    acc += jnp.dot(c_ref[...], w0_ref[88:104, :], preferred_element_type=f32)

    def small(idx_ref, tab_ref, lo, hi, T):
        oh = (idx_ref[...] ==
              lax.broadcasted_iota(jnp.int32, (BLK, T), 1)).astype(f32)
        e = jnp.dot(oh, tab_ref[...], preferred_element_type=f32)
        return jnp.dot(e, w0_ref[lo:hi, :], preferred_element_type=f32)

    acc += small(g_ref, gt_ref, 64, 72, 2)
    acc += small(a_ref, at_ref, 72, 80, 7)
    acc += small(o_ref, ot_ref, 80, 88, 21)
    h = jnp.maximum(acc + b0_ref[...], 0.0)
    h = jnp.maximum(jnp.dot(h, w1_ref[...], preferred_element_type=f32)
                    + b1_ref[...], 0.0)
    h = jnp.maximum(jnp.dot(h, w2_ref[...], preferred_element_type=f32)
                    + b2_ref[...], 0.0)
    out_ref[...] = (jnp.dot(h, w3_ref[...], preferred_element_type=f32)
                    + b3_ref[...])


def _full(shape):
    return pl.BlockSpec(shape, lambda i: (0, 0))


_mlp_call = pl.pallas_call(
    _mlp_body,
    grid=(B // BLK,),
    in_specs=[
        pl.BlockSpec((BLK, D), lambda i: (i, 0)),
        pl.BlockSpec((BLK, D), lambda i: (i, 0)),
        pl.BlockSpec((BLK, 16), lambda i: (i, 0)),
        pl.BlockSpec((BLK, 1), lambda i: (i, 0)),
        pl.BlockSpec((BLK, 1), lambda i: (i, 0)),
        pl.BlockSpec((BLK, 1), lambda i: (i, 0)),
        _full((2, 8)), _full((7, 8)), _full((21, 8)),
        _full((104, 128)), _full((1, 128)),
        _full((128, 64)), _full((1, 64)),
        _full((64, 32)), _full((1, 32)),
        _full((32, 1)), _full((1, 1)),
    ],
    out_specs=pl.BlockSpec((BLK, 1), lambda i: (i, 0)),
    out_shape=jax.ShapeDtypeStruct((B, 1), jnp.float32),
)


def kernel(user_id, movie_id, gender, age, occupation, continuous_features,
           user_table, movie_table, gender_table, age_table, occupation_table,
           W0, b0, W1, b1, W2, b2, W3, b3):
    info = plsc.get_sparse_core_info()
    U, M = _make_gather(info.num_cores, info.num_subcores)(
        user_table, movie_table,
        user_id.astype(jnp.int32), movie_id.astype(jnp.int32))
    return _mlp_call(
        U, M, continuous_features,
        gender.astype(jnp.int32).reshape(B, 1),
        age.astype(jnp.int32).reshape(B, 1),
        occupation.astype(jnp.int32).reshape(B, 1),
        gender_table, age_table, occupation_table,
        W0, b0.reshape(1, 128), W1, b1.reshape(1, 64),
        W2, b2.reshape(1, 32), W3, b3.reshape(1, 1))
